# Initial kernel scaffold; baseline (speedup 1.0000x reference)
import jax
import jax.numpy as jnp
from jax.experimental import pallas as pl


def kernel(feat_id, edge_index, emb, W0, b0, W1, b1):
    raise NotImplementedError("write your pallas kernel here")


# speedup vs baseline: 12.2704x; 12.2704x over previous
import functools

import jax
import jax.numpy as jnp
from jax import lax
from jax.experimental import pallas as pl
from jax.experimental.pallas import tpu as pltpu
from jax.experimental.pallas import tpu_sc as plsc

N = 10000
E = 320000
D = 128
NC, NS = 2, 16
NW = NC * NS
NH = N // 2
JNK = 8
IBLK = 2000
CH = 40
CPB = IBLK // CH
ECT2 = E // NS
NBLK2 = ECT2 // IBLK
ECT = E // NW
DIBLK = 400
NBLK1 = ECT // DIBLK
CCAP = 1240
NCH = 40
BLK = 2000

_mesh = plsc.VectorSubcoreMesh(core_axis_name="c", subcore_axis_name="s",
                               num_cores=NC, num_subcores=NS)


def _sc_deg_gather(src_r, dst_r, feat_r, emb, degh_out, h_out,
                   idx_b, fid_buf, grow, hist, gsem):
    c = lax.axis_index("c")
    s = lax.axis_index("s")
    wid = s * NC + c

    def one_pass(r, p):
        @pl.loop(0, N // 16)
        def _(i):
            hist[pl.ds(i * 16, 16)] = jnp.zeros((16,), jnp.float32)

        @pl.loop(0, NBLK1)
        def _(j):
            pltpu.sync_copy(r.at[pl.ds(wid * ECT + j * DIBLK, DIBLK)], idx_b)

            @pl.loop(0, DIBLK // 16)
            def _(i):
                v = idx_b[pl.ds(i * 16, 16)]
                cnt, last = plsc.scan_count(v)
                plsc.addupdate_scatter(hist, [v], cnt.astype(jnp.float32),
                                       mask=last)

        pltpu.sync_copy(hist, degh_out.at[pl.ds((p * NW + wid) * N, N)])

    one_pass(src_r, 0)
    one_pass(dst_r, 1)

    for k in range(8):
        ci = wid + NW * k

        @pl.when(ci < N // NCH)
        def _():
            pltpu.sync_copy(feat_r.at[pl.ds(ci * NCH, NCH)], fid_buf)
            pltpu.async_copy(emb.at[fid_buf], grow, gsem).wait()
            pltpu.sync_copy(grow, h_out.at[pl.ds(ci * NCH, NCH)])


_deg_gather_call = functools.partial(
    pl.kernel, _sc_deg_gather,
    out_type=(jax.ShapeDtypeStruct((2 * NW * N,), jnp.float32),
              jax.ShapeDtypeStruct((N, D), jnp.float32)),
    mesh=_mesh,
    compiler_params=pltpu.CompilerParams(needs_layout_passes=False),
    scratch_types=[
        pltpu.VMEM((DIBLK,), jnp.int32),
        pltpu.VMEM((NCH,), jnp.int32),
        pltpu.VMEM((NCH, D), jnp.float32),
        pltpu.VMEM((N,), jnp.float32),
        pltpu.SemaphoreType.DMA,
    ],
)()


def _sc_spmm(m_hbm, src_r, dst_r, agg_out,
             src_b, dst_b, csrc, cdst, rows, agg_s, sems):
    c = lax.axis_index("c")
    s = lax.axis_index("s")
    lo = c * NH

    @pl.loop(0, CH)
    def _(i):
        for j in range(D // 16):
            rows[0, i, pl.ds(j * 16, 16)] = jnp.zeros((16,), jnp.float32)

    for k in range(8):
        ci = s + NS * k

        @pl.when(ci < NH // CH)
        def _():
            pltpu.sync_copy(rows.at[0], agg_s.at[pl.ds(ci * CH, CH)])

    plsc.subcore_barrier()
    lane = lax.iota(jnp.int32, 16)

    @pl.loop(0, NBLK2)
    def _(j):
        pltpu.sync_copy(src_r.at[pl.ds(s * ECT2 + j * IBLK, IBLK)], src_b)
        pltpu.sync_copy(dst_r.at[pl.ds(s * ECT2 + j * IBLK, IBLK)], dst_b)

        @pl.loop(0, CCAP // 16)
        def _(i):
            csrc[pl.ds(i * 16, 16)] = lane
            cdst[pl.ds(i * 16, 16)] = NH + (lane & (JNK - 1))

        def cbody(i, off):
            sv = src_b[pl.ds(i * 16, 16)]
            dv = dst_b[pl.ds(i * 16, 16)] - lo
            km = (dv >= 0) & (dv < NH)
            plsc.store_compressed(csrc.at[pl.ds(off, 16)], sv, mask=km)
            plsc.store_compressed(cdst.at[pl.ds(off, 16)], dv, mask=km)
            return off + plsc.all_reduce_population_count(km)[0]

        cnt = pl.loop(0, IBLK // 16, init_carry=jnp.int32(0))(cbody)
        nch = lax.div(cnt + (CH - 1), CH)

        @pl.when(nch > 0)
        def _():
            pltpu.async_copy(m_hbm.at[csrc.at[pl.ds(0, CH)]], rows.at[0],
                             sems.at[0])

        @pl.loop(0, lax.div(nch + 1, 2))
        def _(q):
            g0 = q * 2

            @pl.when(g0 + 1 < nch)
            def _():
                pltpu.async_copy(
                    m_hbm.at[csrc.at[pl.ds((g0 + 1) * CH, CH)]],
                    rows.at[1], sems.at[1])
            pltpu.make_async_copy(
                m_hbm.at[csrc.at[pl.ds(g0 * CH, CH)]], rows.at[0],
                sems.at[0]).wait()
            pltpu.sync_copy(rows.at[0],
                            agg_s.at[cdst.at[pl.ds(g0 * CH, CH)]], add=True)

            @pl.when(g0 + 2 < nch)
            def _():
                pltpu.async_copy(
                    m_hbm.at[csrc.at[pl.ds((g0 + 2) * CH, CH)]],
                    rows.at[0], sems.at[0])

            @pl.when(g0 + 1 < nch)
            def _():
                pltpu.make_async_copy(
                    m_hbm.at[csrc.at[pl.ds((g0 + 1) * CH, CH)]], rows.at[1],
                    sems.at[1]).wait()
                pltpu.sync_copy(rows.at[1],
                                agg_s.at[cdst.at[pl.ds((g0 + 1) * CH, CH)]],
                                add=True)

    plsc.subcore_barrier()
    for k in range(8):
        ci = s + NS * k

        @pl.when(ci < NH // CH)
        def _():
            pltpu.sync_copy(agg_s.at[pl.ds(ci * CH, CH)], rows.at[0])
            pltpu.sync_copy(rows.at[0],
                            agg_out.at[pl.ds(c * NH + ci * CH, CH)])


_spmm_call = functools.partial(
    pl.kernel, _sc_spmm,
    out_type=jax.ShapeDtypeStruct((N, D), jnp.float32),
    mesh=_mesh,
    compiler_params=pltpu.CompilerParams(needs_layout_passes=False),
    scratch_types=[
        pltpu.VMEM((IBLK,), jnp.int32),
        pltpu.VMEM((IBLK,), jnp.int32),
        pltpu.VMEM((CCAP,), jnp.int32),
        pltpu.VMEM((CCAP,), jnp.int32),
        pltpu.VMEM((2, CH, D), jnp.float32),
        pltpu.VMEM_SHARED((NH + JNK, D), jnp.float32),
        pltpu.SemaphoreType.DMA((2,)),
    ],
)()


def _tc_norms(degh_ref, snm_ref, dnm_ref):
    od = jnp.sum(degh_ref[0], axis=0)
    ideg = jnp.sum(degh_ref[1], axis=0)
    snm_ref[...] = jnp.where(od > 0, lax.rsqrt(od), 0.0)[:, None]
    dnm_ref[...] = jnp.where(ideg > 0, lax.rsqrt(ideg), 0.0)[:, None]


def _tc_m0(h_ref, w_ref, snm_ref, m0_ref):
    hw = jnp.dot(h_ref[...], w_ref[...], preferred_element_type=jnp.float32)
    m0_ref[...] = hw * snm_ref[...]


def _tc_layer(agg_ref, snm_ref, dnm_ref, w_ref, b_ref, out_ref):
    h1 = jnp.maximum(agg_ref[...] * dnm_ref[...] + b_ref[...], 0.0)
    out_ref[...] = jnp.dot(h1, w_ref[...],
                           preferred_element_type=jnp.float32) * snm_ref[...]


def _tc_final(agg_ref, dnm_ref, b_ref, out_ref):
    i = pl.program_id(0)
    h2 = jnp.maximum(agg_ref[...] * dnm_ref[...] + b_ref[...], 0.0)
    blkmax = jnp.max(h2, axis=0, keepdims=True)

    @pl.when(i == 0)
    def _():
        out_ref[...] = blkmax

    @pl.when(i > 0)
    def _():
        out_ref[...] = jnp.maximum(out_ref[...], blkmax)


def kernel(feat_id, edge_index, emb, W0, b0, W1, b1):
    src = edge_index[0]
    dst = edge_index[1]

    degh, h = _deg_gather_call(src, dst, feat_id, emb)
    degh = degh.reshape(2, NW, N)

    snm, dnm = pl.pallas_call(
        _tc_norms,
        grid=(1,),
        in_specs=[pl.BlockSpec((2, NW, N), lambda i: (0, 0, 0))],
        out_specs=(pl.BlockSpec((N, 1), lambda i: (0, 0)),
                   pl.BlockSpec((N, 1), lambda i: (0, 0))),
        out_shape=(jax.ShapeDtypeStruct((N, 1), jnp.float32),
                   jax.ShapeDtypeStruct((N, 1), jnp.float32)),
    )(degh)

    m0 = pl.pallas_call(
        _tc_m0,
        grid=(N // BLK,),
        in_specs=[
            pl.BlockSpec((BLK, D), lambda i: (i, 0)),
            pl.BlockSpec((D, D), lambda i: (0, 0)),
            pl.BlockSpec((BLK, 1), lambda i: (i, 0)),
        ],
        out_specs=pl.BlockSpec((BLK, D), lambda i: (i, 0)),
        out_shape=jax.ShapeDtypeStruct((N, D), jnp.float32),
    )(h, W0, snm)

    agg0 = _spmm_call(m0, src, dst)

    m1 = pl.pallas_call(
        _tc_layer,
        grid=(N // BLK,),
        in_specs=[
            pl.BlockSpec((BLK, D), lambda i: (i, 0)),
            pl.BlockSpec((BLK, 1), lambda i: (i, 0)),
            pl.BlockSpec((BLK, 1), lambda i: (i, 0)),
            pl.BlockSpec((D, D), lambda i: (0, 0)),
            pl.BlockSpec((1, D), lambda i: (0, 0)),
        ],
        out_specs=pl.BlockSpec((BLK, D), lambda i: (i, 0)),
        out_shape=jax.ShapeDtypeStruct((N, D), jnp.float32),
    )(agg0, snm, dnm, W1, b0.reshape(1, D))

    agg1 = _spmm_call(m1, src, dst)

    out = pl.pallas_call(
        _tc_final,
        grid=(N // BLK,),
        in_specs=[
            pl.BlockSpec((BLK, D), lambda i: (i, 0)),
            pl.BlockSpec((BLK, 1), lambda i: (i, 0)),
            pl.BlockSpec((1, D), lambda i: (0, 0)),
        ],
        out_specs=pl.BlockSpec((1, D), lambda i: (0, 0)),
        out_shape=jax.ShapeDtypeStruct((1, D), jnp.float32),
    )(agg1, dnm, b1.reshape(1, D))

    return out
